# Initial kernel scaffold; baseline (speedup 1.0000x reference)
import jax
import jax.numpy as jnp
from jax.experimental import pallas as pl


def kernel(x, W1, b1, W2, b2, edge_index):
    raise NotImplementedError("write your pallas kernel here")


# speedup vs baseline: 14.9477x; 14.9477x over previous
import functools

import jax
import jax.numpy as jnp
from jax import lax
from jax.experimental import pallas as pl
from jax.experimental.pallas import tpu as pltpu
from jax.experimental.pallas import tpu_sc as plsc

N_NODES = 10000
N_EDGES = 320000
D = 128
DH = D // 2

NC = 2
NS = 16
LANES = 16

E_PER_TILE = N_EDGES // NS
CHUNK = 80
N_CHUNKS = E_PER_TILE // CHUNK
NBUF = 2
ROWS_MAIN = 632
ROWS_LAST = N_NODES - (NS - 1) * ROWS_MAIN

_mesh = plsc.VectorSubcoreMesh(core_axis_name="c", subcore_axis_name="s")


@functools.partial(
    pl.kernel,
    out_type=[
        jax.ShapeDtypeStruct((NC, N_NODES, DH), jnp.float32),
        jax.ShapeDtypeStruct((NS, N_NODES), jnp.float32),
        jax.ShapeDtypeStruct((NS, N_NODES), jnp.float32),
    ],
    mesh=_mesh,
    compiler_params=pltpu.CompilerParams(needs_layout_passes=False,
                                         use_tc_tiling_on_sc=False),
    scratch_types=[
        pltpu.VMEM((N_CHUNKS, CHUNK), jnp.int32),
        pltpu.VMEM((N_CHUNKS, CHUNK), jnp.int32),
        pltpu.VMEM((NBUF, CHUNK, DH), jnp.float32),
        pltpu.VMEM((N_NODES,), jnp.float32),
        pltpu.VMEM((N_NODES,), jnp.float32),
        pltpu.VMEM_SHARED((N_NODES, DH), jnp.float32),
        pltpu.SemaphoreType.DMA,
        pltpu.SemaphoreType.DMA,
    ],
)
def _sc_accumulate(x0_hbm, x1_hbm, src_hbm, dst_hbm, zrows_hbm,
                   agg_out, odeg_out, ideg_out,
                   src_v, dst_v, rows_v, odeg_v, ideg_v,
                   agg_sh, sem0, sem1):
    cid = lax.axis_index("c")
    sid = lax.axis_index("s")
    sems = (sem0, sem1)

    pltpu.sync_copy(src_hbm.at[sid], src_v)
    pltpu.sync_copy(dst_hbm.at[sid], dst_v)

    @pl.when(sid < NS - 1)
    def _():
        nslc = pl.ds(pl.multiple_of(sid * ROWS_MAIN, 8), ROWS_MAIN)
        pltpu.sync_copy(zrows_hbm, agg_sh.at[nslc])

    @pl.when(sid == NS - 1)
    def _():
        nslc = pl.ds((NS - 1) * ROWS_MAIN, ROWS_LAST)
        pltpu.sync_copy(zrows_hbm.at[pl.ds(0, ROWS_LAST)], agg_sh.at[nslc])

    @pl.when(cid == 0)
    def _():
        zv = jnp.zeros((LANES,), jnp.float32)

        def zbody(i, carry):
            off = pl.multiple_of(i * LANES, LANES)
            odeg_v[pl.ds(off, LANES)] = zv
            ideg_v[pl.ds(off, LANES)] = zv
            return carry

        lax.fori_loop(0, N_NODES // LANES, zbody, None)

    plsc.subcore_barrier()

    ones16 = jnp.ones((LANES,), jnp.float32)

    def pipeline(x_hbm, do_hist):
        def consume(g, b):
            pltpu.make_async_copy(x_hbm.at[src_v.at[g]], rows_v.at[b],
                                  sems[b]).wait()
            pltpu.sync_copy(rows_v.at[b], agg_sh.at[dst_v.at[g]], add=True)
            if do_hist:
                for c in range(CHUNK // LANES):
                    sl = pl.ds(c * LANES, LANES)
                    plsc.addupdate_scatter(odeg_v, [src_v.at[g][sl]], ones16)
                    plsc.addupdate_scatter(ideg_v, [dst_v.at[g][sl]], ones16)

        for b in range(NBUF):
            pltpu.async_copy(x_hbm.at[src_v.at[b]], rows_v.at[b], sems[b])

        def outer(o, carry):
            for b in range(NBUF):
                g = o * NBUF + b
                consume(g, b)
                nxt = g + NBUF

                @pl.when(nxt < N_CHUNKS)
                def _():
                    pltpu.async_copy(x_hbm.at[src_v.at[nxt]], rows_v.at[b],
                                     sems[b])
            return carry

        lax.fori_loop(0, N_CHUNKS // NBUF, outer, None)
        for g in range(N_CHUNKS - N_CHUNKS % NBUF, N_CHUNKS):
            consume(g, g % NBUF)

    @pl.when(cid == 0)
    def _():
        pipeline(x0_hbm, do_hist=True)

    @pl.when(cid == 1)
    def _():
        pipeline(x1_hbm, do_hist=False)

    plsc.subcore_barrier()

    @pl.when(cid == 0)
    def _():
        pltpu.sync_copy(odeg_v, odeg_out.at[sid])
        pltpu.sync_copy(ideg_v, ideg_out.at[sid])

    @pl.when(sid < NS - 1)
    def _():
        nslc = pl.ds(pl.multiple_of(sid * ROWS_MAIN, 8), ROWS_MAIN)
        pltpu.sync_copy(agg_sh.at[nslc], agg_out.at[cid, nslc])

    @pl.when(sid == NS - 1)
    def _():
        nslc = pl.ds((NS - 1) * ROWS_MAIN, ROWS_LAST)
        pltpu.sync_copy(agg_sh.at[nslc], agg_out.at[cid, nslc])


def _tc_finish_body(aggL, aggR, odeg, ideg, w1L, w1R, b1, w2, b2,
                    pooled_ref, logp_ref):
    ones_ns = jnp.ones((NS, 1), jnp.float32)
    ideg_col = lax.dot_general(ideg[...], ones_ns, (((0,), (0,)), ((), ())),
                               preferred_element_type=jnp.float32)
    h = lax.dot_general(aggL[...], w1L[...], (((1,), (1,)), ((), ())),
                        preferred_element_type=jnp.float32)
    h = h + lax.dot_general(aggR[...], w1R[...], (((1,), (1,)), ((), ())),
                            preferred_element_type=jnp.float32)
    u = jnp.maximum(h + ideg_col * b1[...], 0.0)
    s_parts = lax.dot_general(odeg[...], u, (((1,), (0,)), ((), ())),
                              preferred_element_type=jnp.float32)
    s = jnp.sum(s_parts, axis=0, keepdims=True)
    pooled = lax.dot_general(s, w2[...], (((1,), (1,)), ((), ())),
                             preferred_element_type=jnp.float32)
    pooled = pooled + float(N_EDGES) * b2[...]
    pooled_ref[...] = pooled
    m = jnp.max(pooled, axis=1, keepdims=True)
    lse = m + jnp.log(jnp.sum(jnp.exp(pooled - m), axis=1, keepdims=True))
    logp_ref[...] = pooled - lse


_tc_finish = pl.pallas_call(
    _tc_finish_body,
    out_shape=(
        jax.ShapeDtypeStruct((1, D), jnp.float32),
        jax.ShapeDtypeStruct((1, D), jnp.float32),
    ),
)


@jax.jit
def kernel(x, W1, b1, W2, b2, edge_index):
    src = edge_index[0].reshape(NS, N_CHUNKS, CHUNK)
    dst = edge_index[1].reshape(NS, N_CHUNKS, CHUNK)
    x0 = x[:, :DH]
    x1 = x[:, DH:]
    zrows = jnp.zeros((ROWS_MAIN, DH), jnp.float32)
    agg_p, odeg_p, ideg_p = _sc_accumulate(x0, x1, src, dst, zrows)
    pooled, logp = _tc_finish(agg_p[0], agg_p[1], odeg_p, ideg_p,
                              W1[:, :DH], W1[:, DH:],
                              b1.reshape(1, D), W2, b2.reshape(1, D))
    return (pooled, logp)


# speedup vs baseline: 21.5607x; 1.4424x over previous
import functools

import jax
import jax.numpy as jnp
from jax import lax
from jax.experimental import pallas as pl
from jax.experimental.pallas import tpu as pltpu
from jax.experimental.pallas import tpu_sc as plsc

N_NODES = 10000
N_EDGES = 320000
D = 128
DH = D // 2

NC = 2
NS = 16
LANES = 16

E_PER_TILE = N_EDGES // NS
CHUNK = 80
N_CHUNKS = E_PER_TILE // CHUNK
NBUF = 4
ROWS_MAIN = 632
ROWS_LAST = N_NODES - (NS - 1) * ROWS_MAIN

_mesh = plsc.VectorSubcoreMesh(core_axis_name="c", subcore_axis_name="s")


@functools.partial(
    pl.kernel,
    out_type=[
        jax.ShapeDtypeStruct((NC, N_NODES, DH), jnp.float32),
        jax.ShapeDtypeStruct((NC, NS, N_NODES), jnp.float32),
    ],
    mesh=_mesh,
    compiler_params=pltpu.CompilerParams(needs_layout_passes=False,
                                         use_tc_tiling_on_sc=False),
    scratch_types=[
        pltpu.VMEM((N_CHUNKS, CHUNK), jnp.int32),
        pltpu.VMEM((N_CHUNKS, CHUNK), jnp.int32),
        pltpu.VMEM((N_CHUNKS, CHUNK), jnp.int32),
        pltpu.VMEM((NBUF, CHUNK, DH), jnp.float32),
        pltpu.VMEM((N_NODES,), jnp.float32),
        pltpu.VMEM_SHARED((N_NODES, DH), jnp.float32),
        [pltpu.SemaphoreType.DMA] * NBUF,
        [pltpu.SemaphoreType.DMA] * NBUF,
    ],
)
def _sc_accumulate(x2_hbm, src_hbm, dst_hbm, zrows_hbm,
                   agg_out, deg_out,
                   src_v, dst_v, gsrc_v, rows_v, deg_v, agg_sh,
                   gsems, ssems):
    cid = lax.axis_index("c")
    sid = lax.axis_index("s")

    pltpu.sync_copy(src_hbm.at[sid], src_v)
    pltpu.sync_copy(dst_hbm.at[sid], dst_v)

    twos = jnp.full((LANES,), 2, jnp.int32)
    cidv = jnp.full((LANES,), 1, jnp.int32) * cid
    SUBV = CHUNK // LANES

    def gidx_body(i, carry):
        r = i // SUBV
        sl = pl.ds((i % SUBV) * LANES, LANES)
        gsrc_v.at[r][sl] = src_v.at[r][sl] * twos + cidv
        return carry

    lax.fori_loop(0, N_CHUNKS * SUBV, gidx_body, None)

    @pl.when(sid < NS - 1)
    def _():
        nslc = pl.ds(pl.multiple_of(sid * ROWS_MAIN, 8), ROWS_MAIN)
        pltpu.sync_copy(zrows_hbm, agg_sh.at[nslc])

    @pl.when(sid == NS - 1)
    def _():
        nslc = pl.ds((NS - 1) * ROWS_MAIN, ROWS_LAST)
        pltpu.sync_copy(zrows_hbm.at[pl.ds(0, ROWS_LAST)], agg_sh.at[nslc])

    zv = jnp.zeros((LANES,), jnp.float32)

    def zbody(i, carry):
        deg_v[pl.ds(pl.multiple_of(i * LANES, LANES), LANES)] = zv
        return carry

    lax.fori_loop(0, N_NODES // LANES, zbody, None)

    plsc.subcore_barrier()

    ones16 = jnp.ones((LANES,), jnp.float32)

    def gather(g, b):
        pltpu.async_copy(x2_hbm.at[gsrc_v.at[g]], rows_v.at[b], gsems[b])

    def wait_gather(g, b):
        pltpu.make_async_copy(x2_hbm.at[gsrc_v.at[g]], rows_v.at[b],
                              gsems[b]).wait()

    def scatter(g, b):
        pltpu.async_copy(rows_v.at[b], agg_sh.at[dst_v.at[g]], ssems[b],
                         add=True)

    def wait_scatter(g, b):
        pltpu.make_async_copy(rows_v.at[b], agg_sh.at[dst_v.at[g]],
                              ssems[b]).wait()

    def pipeline(idx_ref):
        def hist(g):
            for c in range(SUBV):
                sl = pl.ds(c * LANES, LANES)
                plsc.addupdate_scatter(deg_v, [idx_ref.at[g][sl]], ones16)

        def round_(g, bmod, drain=True, prefetch=True):
            if drain:
                wait_scatter(g - 2, (bmod + 2) % NBUF)
            if prefetch:
                gather(g + 2, (bmod + 2) % NBUF)
            wait_gather(g, bmod)
            scatter(g, bmod)
            hist(g)

        gather(0, 0)
        gather(1, 1)
        round_(0, 0, drain=False)
        round_(1, 1, drain=False)

        def outer(o, carry):
            for b in range(NBUF):
                round_(o * NBUF + 2 + b, (2 + b) % NBUF)
            return carry

        lax.fori_loop(0, (N_CHUNKS - 2 - 4) // NBUF, outer, None)
        round_(246, 2)
        round_(247, 3)
        round_(248, 0, prefetch=False)
        round_(249, 1, prefetch=False)
        wait_scatter(248, 0)
        wait_scatter(249, 1)

    @pl.when(cid == 0)
    def _():
        pipeline(src_v)

    @pl.when(cid == 1)
    def _():
        pipeline(dst_v)

    plsc.subcore_barrier()

    pltpu.sync_copy(deg_v, deg_out.at[cid, sid])

    @pl.when(sid < NS - 1)
    def _():
        nslc = pl.ds(pl.multiple_of(sid * ROWS_MAIN, 8), ROWS_MAIN)
        pltpu.sync_copy(agg_sh.at[nslc], agg_out.at[cid, nslc])

    @pl.when(sid == NS - 1)
    def _():
        nslc = pl.ds((NS - 1) * ROWS_MAIN, ROWS_LAST)
        pltpu.sync_copy(agg_sh.at[nslc], agg_out.at[cid, nslc])


def _tc_finish_body(agg, deg, w1, b1, w2, b2, pooled_ref, logp_ref):
    ones_ns = jnp.ones((NS, 1), jnp.float32)
    ideg_col = lax.dot_general(deg[1], ones_ns, (((0,), (0,)), ((), ())),
                               preferred_element_type=jnp.float32)
    h = lax.dot_general(agg[0], w1[:, :DH], (((1,), (1,)), ((), ())),
                        preferred_element_type=jnp.float32)
    h = h + lax.dot_general(agg[1], w1[:, DH:], (((1,), (1,)), ((), ())),
                            preferred_element_type=jnp.float32)
    u = jnp.maximum(h + ideg_col * b1[...], 0.0)
    s_parts = lax.dot_general(deg[0], u, (((1,), (0,)), ((), ())),
                              preferred_element_type=jnp.float32)
    s = jnp.sum(s_parts, axis=0, keepdims=True)
    pooled = lax.dot_general(s, w2[...], (((1,), (1,)), ((), ())),
                             preferred_element_type=jnp.float32)
    pooled = pooled + float(N_EDGES) * b2[...]
    pooled_ref[...] = pooled
    m = jnp.max(pooled, axis=1, keepdims=True)
    lse = m + jnp.log(jnp.sum(jnp.exp(pooled - m), axis=1, keepdims=True))
    logp_ref[...] = pooled - lse


_tc_finish = pl.pallas_call(
    _tc_finish_body,
    out_shape=(
        jax.ShapeDtypeStruct((1, D), jnp.float32),
        jax.ShapeDtypeStruct((1, D), jnp.float32),
    ),
)


@jax.jit
def kernel(x, W1, b1, W2, b2, edge_index):
    src = edge_index[0].reshape(NS, N_CHUNKS, CHUNK)
    dst = edge_index[1].reshape(NS, N_CHUNKS, CHUNK)
    x2 = x.reshape(2 * N_NODES, DH)
    zrows = jnp.zeros((ROWS_MAIN, DH), jnp.float32)
    agg_p, deg_p = _sc_accumulate(x2, src, dst, zrows)
    pooled, logp = _tc_finish(agg_p, deg_p, W1, b1.reshape(1, D),
                              W2, b2.reshape(1, D))
    return (pooled, logp)


# speedup vs baseline: 23.9644x; 1.1115x over previous
import functools

import jax
import jax.numpy as jnp
from jax import lax
from jax.experimental import pallas as pl
from jax.experimental.pallas import tpu as pltpu
from jax.experimental.pallas import tpu_sc as plsc

N_NODES = 10000
N_EDGES = 320000
D = 128
DH = D // 2

NC = 2
NS = 16
LANES = 16

E_PER_TILE = N_EDGES // NS
CHUNK = 80
N_CHUNKS = E_PER_TILE // CHUNK
NBUF = 4
ROWS_MAIN = 632
ROWS_LAST = N_NODES - (NS - 1) * ROWS_MAIN

_mesh = plsc.VectorSubcoreMesh(core_axis_name="c", subcore_axis_name="s")


@functools.partial(
    pl.kernel,
    out_type=[
        jax.ShapeDtypeStruct((NC, N_NODES, DH), jnp.bfloat16),
        jax.ShapeDtypeStruct((NC, NS, N_NODES), jnp.float32),
    ],
    mesh=_mesh,
    compiler_params=pltpu.CompilerParams(needs_layout_passes=False,
                                         use_tc_tiling_on_sc=False),
    scratch_types=[
        pltpu.VMEM((N_CHUNKS, CHUNK), jnp.int32),
        pltpu.VMEM((N_CHUNKS, CHUNK), jnp.int32),
        pltpu.VMEM((N_CHUNKS, CHUNK), jnp.int32),
        pltpu.VMEM((NBUF, CHUNK, DH), jnp.bfloat16),
        pltpu.VMEM((N_NODES,), jnp.float32),
        pltpu.VMEM_SHARED((N_NODES, DH), jnp.bfloat16),
        [pltpu.SemaphoreType.DMA] * NBUF,
        [pltpu.SemaphoreType.DMA] * NBUF,
    ],
)
def _sc_accumulate(x2_hbm, src_hbm, dst_hbm, zrows_hbm,
                   agg_out, deg_out,
                   src_v, dst_v, gsrc_v, rows_v, deg_v, agg_sh,
                   gsems, ssems):
    cid = lax.axis_index("c")
    sid = lax.axis_index("s")

    pltpu.sync_copy(src_hbm.at[sid], src_v)
    pltpu.sync_copy(dst_hbm.at[sid], dst_v)

    twos = jnp.full((LANES,), 2, jnp.int32)
    cidv = jnp.full((LANES,), 1, jnp.int32) * cid
    SUBV = CHUNK // LANES

    def gidx_body(i, carry):
        r = i // SUBV
        sl = pl.ds((i % SUBV) * LANES, LANES)
        gsrc_v.at[r][sl] = src_v.at[r][sl] * twos + cidv
        return carry

    lax.fori_loop(0, N_CHUNKS * SUBV, gidx_body, None)

    @pl.when(sid < NS - 1)
    def _():
        nslc = pl.ds(pl.multiple_of(sid * ROWS_MAIN, 8), ROWS_MAIN)
        pltpu.sync_copy(zrows_hbm, agg_sh.at[nslc])

    @pl.when(sid == NS - 1)
    def _():
        nslc = pl.ds((NS - 1) * ROWS_MAIN, ROWS_LAST)
        pltpu.sync_copy(zrows_hbm.at[pl.ds(0, ROWS_LAST)], agg_sh.at[nslc])

    zv = jnp.zeros((LANES,), jnp.float32)

    def zbody(i, carry):
        deg_v[pl.ds(pl.multiple_of(i * LANES, LANES), LANES)] = zv
        return carry

    lax.fori_loop(0, N_NODES // LANES, zbody, None)

    plsc.subcore_barrier()

    ones16 = jnp.ones((LANES,), jnp.float32)

    def gather(g, b):
        pltpu.async_copy(x2_hbm.at[gsrc_v.at[g]], rows_v.at[b], gsems[b])

    def wait_gather(g, b):
        pltpu.make_async_copy(x2_hbm.at[gsrc_v.at[g]], rows_v.at[b],
                              gsems[b]).wait()

    def scatter(g, b):
        pltpu.async_copy(rows_v.at[b], agg_sh.at[dst_v.at[g]], ssems[b],
                         add=True)

    def wait_scatter(g, b):
        pltpu.make_async_copy(rows_v.at[b], agg_sh.at[dst_v.at[g]],
                              ssems[b]).wait()

    def pipeline(idx_ref):
        def hist(g):
            for c in range(SUBV):
                sl = pl.ds(c * LANES, LANES)
                plsc.addupdate_scatter(deg_v, [idx_ref.at[g][sl]], ones16)

        def round_(g, bmod, drain=True, prefetch=True):
            if drain:
                wait_scatter(g - 2, (bmod + 2) % NBUF)
            if prefetch:
                gather(g + 2, (bmod + 2) % NBUF)
            wait_gather(g, bmod)
            scatter(g, bmod)
            hist(g)

        gather(0, 0)
        gather(1, 1)
        round_(0, 0, drain=False)
        round_(1, 1, drain=False)

        def outer(o, carry):
            for b in range(NBUF):
                round_(o * NBUF + 2 + b, (2 + b) % NBUF)
            return carry

        lax.fori_loop(0, (N_CHUNKS - 2 - 4) // NBUF, outer, None)
        round_(246, 2)
        round_(247, 3)
        round_(248, 0, prefetch=False)
        round_(249, 1, prefetch=False)
        wait_scatter(248, 0)
        wait_scatter(249, 1)

    @pl.when(cid == 0)
    def _():
        pipeline(src_v)

    @pl.when(cid == 1)
    def _():
        pipeline(dst_v)

    plsc.subcore_barrier()

    pltpu.sync_copy(deg_v, deg_out.at[cid, sid])

    @pl.when(sid < NS - 1)
    def _():
        nslc = pl.ds(pl.multiple_of(sid * ROWS_MAIN, 8), ROWS_MAIN)
        pltpu.sync_copy(agg_sh.at[nslc], agg_out.at[cid, nslc])

    @pl.when(sid == NS - 1)
    def _():
        nslc = pl.ds((NS - 1) * ROWS_MAIN, ROWS_LAST)
        pltpu.sync_copy(agg_sh.at[nslc], agg_out.at[cid, nslc])


def _tc_finish_body(agg, deg, w1, b1, w2, b2, pooled_ref, logp_ref):
    ones_ns = jnp.ones((NS, 1), jnp.float32)
    ideg_col = lax.dot_general(deg[1], ones_ns, (((0,), (0,)), ((), ())),
                               preferred_element_type=jnp.float32)
    h = lax.dot_general(agg[0].astype(jnp.float32), w1[:, :DH], (((1,), (1,)), ((), ())),
                        preferred_element_type=jnp.float32)
    h = h + lax.dot_general(agg[1].astype(jnp.float32), w1[:, DH:], (((1,), (1,)), ((), ())),
                            preferred_element_type=jnp.float32)
    u = jnp.maximum(h + ideg_col * b1[...], 0.0)
    s_parts = lax.dot_general(deg[0], u, (((1,), (0,)), ((), ())),
                              preferred_element_type=jnp.float32)
    s = jnp.sum(s_parts, axis=0, keepdims=True)
    pooled = lax.dot_general(s, w2[...], (((1,), (1,)), ((), ())),
                             preferred_element_type=jnp.float32)
    pooled = pooled + float(N_EDGES) * b2[...]
    pooled_ref[...] = pooled
    m = jnp.max(pooled, axis=1, keepdims=True)
    lse = m + jnp.log(jnp.sum(jnp.exp(pooled - m), axis=1, keepdims=True))
    logp_ref[...] = pooled - lse


_tc_finish = pl.pallas_call(
    _tc_finish_body,
    out_shape=(
        jax.ShapeDtypeStruct((1, D), jnp.float32),
        jax.ShapeDtypeStruct((1, D), jnp.float32),
    ),
)


@jax.jit
def kernel(x, W1, b1, W2, b2, edge_index):
    src = edge_index[0].reshape(NS, N_CHUNKS, CHUNK)
    dst = edge_index[1].reshape(NS, N_CHUNKS, CHUNK)
    x2 = x.astype(jnp.bfloat16).reshape(2 * N_NODES, DH)
    zrows = jnp.zeros((ROWS_MAIN, DH), jnp.bfloat16)
    agg_p, deg_p = _sc_accumulate(x2, src, dst, zrows)
    pooled, logp = _tc_finish(agg_p, deg_p, W1, b1.reshape(1, D),
                              W2, b2.reshape(1, D))
    return (pooled, logp)


# speedup vs baseline: 25.9253x; 1.0818x over previous
import functools

import jax
import jax.numpy as jnp
from jax import lax
from jax.experimental import pallas as pl
from jax.experimental.pallas import tpu as pltpu
from jax.experimental.pallas import tpu_sc as plsc

N_NODES = 10000
N_EDGES = 320000
D = 128
DH = D // 2

NC = 2
NS = 16
LANES = 16

E_PER_TILE = N_EDGES // NS
CHUNK = 80
N_CHUNKS = E_PER_TILE // CHUNK
NBUF = 4
ROWS_MAIN = 632
ROWS_LAST = N_NODES - (NS - 1) * ROWS_MAIN

_mesh = plsc.VectorSubcoreMesh(core_axis_name="c", subcore_axis_name="s")


@functools.partial(
    pl.kernel,
    out_type=[
        jax.ShapeDtypeStruct((N_NODES, D), jnp.bfloat16),
        jax.ShapeDtypeStruct((NC, NS, N_NODES), jnp.float32),
    ],
    mesh=_mesh,
    compiler_params=pltpu.CompilerParams(needs_layout_passes=False,
                                         use_tc_tiling_on_sc=False),
    scratch_types=[
        pltpu.VMEM((E_PER_TILE,), jnp.int32),
        pltpu.VMEM((E_PER_TILE,), jnp.int32),
        pltpu.VMEM((NBUF, CHUNK, DH), jnp.bfloat16),
        pltpu.VMEM((N_NODES,), jnp.float32),
        pltpu.VMEM_SHARED((N_NODES, DH), jnp.bfloat16),
        [pltpu.SemaphoreType.DMA] * NBUF,
        [pltpu.SemaphoreType.DMA] * NBUF,
    ],
)
def _sc_accumulate(x0_hbm, x1_hbm, edge_hbm, zrows_hbm,
                   agg_out, deg_out,
                   src_v, dst_v, rows_v, deg_v, agg_sh,
                   gsems, ssems):
    cid = lax.axis_index("c")
    sid = lax.axis_index("s")

    eslc = pl.ds(pl.multiple_of(sid * E_PER_TILE, 8), E_PER_TILE)
    pltpu.sync_copy(edge_hbm.at[0, eslc], src_v)
    pltpu.sync_copy(edge_hbm.at[1, eslc], dst_v)

    @pl.when(sid < NS - 1)
    def _():
        nslc = pl.ds(pl.multiple_of(sid * ROWS_MAIN, 8), ROWS_MAIN)
        pltpu.sync_copy(zrows_hbm, agg_sh.at[nslc])

    @pl.when(sid == NS - 1)
    def _():
        nslc = pl.ds((NS - 1) * ROWS_MAIN, ROWS_LAST)
        pltpu.sync_copy(zrows_hbm.at[pl.ds(0, ROWS_LAST)], agg_sh.at[nslc])

    zv = jnp.zeros((LANES,), jnp.float32)

    def zbody(i, carry):
        deg_v[pl.ds(pl.multiple_of(i * LANES, LANES), LANES)] = zv
        return carry

    lax.fori_loop(0, N_NODES // LANES, zbody, None)

    plsc.subcore_barrier()

    ones16 = jnp.ones((LANES,), jnp.float32)
    SUBV = CHUNK // LANES

    def pipeline(x_hbm, idx_ref):
        def chunk_of(g):
            return pl.ds(pl.multiple_of(g * CHUNK, 8), CHUNK)

        def gather(g, b):
            pltpu.async_copy(x_hbm.at[src_v.at[chunk_of(g)]], rows_v.at[b],
                             gsems[b])

        def wait_gather(g, b):
            pltpu.make_async_copy(x_hbm.at[src_v.at[chunk_of(g)]],
                                  rows_v.at[b], gsems[b]).wait()

        def scatter(g, b):
            pltpu.async_copy(rows_v.at[b], agg_sh.at[dst_v.at[chunk_of(g)]],
                             ssems[b], add=True)

        def wait_scatter(g, b):
            pltpu.make_async_copy(rows_v.at[b],
                                  agg_sh.at[dst_v.at[chunk_of(g)]],
                                  ssems[b]).wait()

        def hist(g):
            for c in range(SUBV):
                sl = pl.ds(pl.multiple_of(g * CHUNK, 8) + c * LANES, LANES)
                plsc.addupdate_scatter(deg_v, [idx_ref[sl]], ones16)

        def round_(g, bmod, drain=True, prefetch=True):
            if drain:
                wait_scatter(g - 2, (bmod + 2) % NBUF)
            if prefetch:
                gather(g + 2, (bmod + 2) % NBUF)
            wait_gather(g, bmod)
            scatter(g, bmod)
            hist(g)

        gather(0, 0)
        gather(1, 1)
        round_(0, 0, drain=False)
        round_(1, 1, drain=False)

        def outer(o, carry):
            for b in range(NBUF):
                round_(o * NBUF + 2 + b, (2 + b) % NBUF)
            return carry

        lax.fori_loop(0, (N_CHUNKS - 2 - 4) // NBUF, outer, None)
        round_(246, 2)
        round_(247, 3)
        round_(248, 0, prefetch=False)
        round_(249, 1, prefetch=False)
        wait_scatter(248, 0)
        wait_scatter(249, 1)

    @pl.when(cid == 0)
    def _():
        pipeline(x0_hbm, src_v)

    @pl.when(cid == 1)
    def _():
        pipeline(x1_hbm, dst_v)

    plsc.subcore_barrier()

    pltpu.sync_copy(deg_v, deg_out.at[cid, sid])
    cslc = pl.ds(pl.multiple_of(cid * DH, 8), DH)

    @pl.when(sid < NS - 1)
    def _():
        nslc = pl.ds(pl.multiple_of(sid * ROWS_MAIN, 8), ROWS_MAIN)
        pltpu.sync_copy(agg_sh.at[nslc], agg_out.at[nslc, cslc])

    @pl.when(sid == NS - 1)
    def _():
        nslc = pl.ds((NS - 1) * ROWS_MAIN, ROWS_LAST)
        pltpu.sync_copy(agg_sh.at[nslc], agg_out.at[nslc, cslc])


def _tc_finish_body(agg, deg, w1, b1, w2, b2, pooled_ref, logp_ref):
    ones_ns = jnp.ones((NS, 1), jnp.float32)
    ideg_col = lax.dot_general(deg[1], ones_ns, (((0,), (0,)), ((), ())),
                               preferred_element_type=jnp.float32)
    h = lax.dot_general(agg[...].astype(jnp.float32), w1[...],
                        (((1,), (1,)), ((), ())),
                        preferred_element_type=jnp.float32)
    u = jnp.maximum(h + ideg_col * b1[...], 0.0)
    s_parts = lax.dot_general(deg[0], u, (((1,), (0,)), ((), ())),
                              preferred_element_type=jnp.float32)
    s = jnp.sum(s_parts, axis=0, keepdims=True)
    pooled = lax.dot_general(s, w2[...], (((1,), (1,)), ((), ())),
                             preferred_element_type=jnp.float32)
    pooled = pooled + float(N_EDGES) * b2[...]
    pooled_ref[...] = pooled
    m = jnp.max(pooled, axis=1, keepdims=True)
    lse = m + jnp.log(jnp.sum(jnp.exp(pooled - m), axis=1, keepdims=True))
    logp_ref[...] = pooled - lse


_tc_finish = pl.pallas_call(
    _tc_finish_body,
    out_shape=(
        jax.ShapeDtypeStruct((1, D), jnp.float32),
        jax.ShapeDtypeStruct((1, D), jnp.float32),
    ),
)


@jax.jit
def kernel(x, W1, b1, W2, b2, edge_index):
    xbf = x.astype(jnp.bfloat16)
    x0 = xbf[:, :DH]
    x1 = xbf[:, DH:]
    zrows = jnp.zeros((ROWS_MAIN, DH), jnp.bfloat16)
    agg, deg_p = _sc_accumulate(x0, x1, edge_index, zrows)
    pooled, logp = _tc_finish(agg, deg_p, W1, b1.reshape(1, D),
                              W2, b2.reshape(1, D))
    return (pooled, logp)


# speedup vs baseline: 31.5506x; 1.2170x over previous
import functools

import jax
import jax.numpy as jnp
from jax import lax
from jax.experimental import pallas as pl
from jax.experimental.pallas import tpu as pltpu
from jax.experimental.pallas import tpu_sc as plsc

N_NODES = 10000
N_EDGES = 320000
D = 128
DH = D // 2

NC = 2
NS = 16
LANES = 16

E_PER_TILE = N_EDGES // NS
CHUNK = 400
N_CHUNKS = E_PER_TILE // CHUNK
NBUF = 4
ROWS_MAIN = 632
ROWS_LAST = N_NODES - (NS - 1) * ROWS_MAIN

_mesh = plsc.VectorSubcoreMesh(core_axis_name="c", subcore_axis_name="s")


@functools.partial(
    pl.kernel,
    out_type=[
        jax.ShapeDtypeStruct((N_NODES, D), jnp.bfloat16),
        jax.ShapeDtypeStruct((NC, NS, N_NODES), jnp.float32),
    ],
    mesh=_mesh,
    compiler_params=pltpu.CompilerParams(needs_layout_passes=False,
                                         use_tc_tiling_on_sc=False),
    scratch_types=[
        pltpu.VMEM((E_PER_TILE,), jnp.int32),
        pltpu.VMEM((E_PER_TILE,), jnp.int32),
        pltpu.VMEM((NBUF, CHUNK, DH), jnp.bfloat16),
        pltpu.VMEM((N_NODES,), jnp.float32),
        pltpu.VMEM_SHARED((N_NODES, DH), jnp.bfloat16),
        [pltpu.SemaphoreType.DMA] * NBUF,
        [pltpu.SemaphoreType.DMA] * NBUF,
    ],
)
def _sc_accumulate(x0_hbm, x1_hbm, edge_hbm, zrows_hbm,
                   agg_out, deg_out,
                   src_v, dst_v, rows_v, deg_v, agg_sh,
                   gsems, ssems):
    cid = lax.axis_index("c")
    sid = lax.axis_index("s")

    eslc = pl.ds(pl.multiple_of(sid * E_PER_TILE, 8), E_PER_TILE)
    pltpu.sync_copy(edge_hbm.at[0, eslc], src_v)
    pltpu.sync_copy(edge_hbm.at[1, eslc], dst_v)

    @pl.when(sid < NS - 1)
    def _():
        nslc = pl.ds(pl.multiple_of(sid * ROWS_MAIN, 8), ROWS_MAIN)
        pltpu.sync_copy(zrows_hbm, agg_sh.at[nslc])

    @pl.when(sid == NS - 1)
    def _():
        nslc = pl.ds((NS - 1) * ROWS_MAIN, ROWS_LAST)
        pltpu.sync_copy(zrows_hbm.at[pl.ds(0, ROWS_LAST)], agg_sh.at[nslc])

    zv = jnp.zeros((LANES,), jnp.float32)

    def zbody(i, carry):
        deg_v[pl.ds(pl.multiple_of(i * LANES, LANES), LANES)] = zv
        return carry

    lax.fori_loop(0, N_NODES // LANES, zbody, None)

    plsc.subcore_barrier()

    ones16 = jnp.ones((LANES,), jnp.float32)
    SUBV = CHUNK // LANES

    def pipeline(x_hbm, idx_ref):
        def chunk_of(g):
            return pl.ds(pl.multiple_of(g * CHUNK, 8), CHUNK)

        def gather(g, b):
            pltpu.async_copy(x_hbm.at[src_v.at[chunk_of(g)]], rows_v.at[b],
                             gsems[b])

        def wait_gather(g, b):
            pltpu.make_async_copy(x_hbm.at[src_v.at[chunk_of(g)]],
                                  rows_v.at[b], gsems[b]).wait()

        def scatter(g, b):
            pltpu.async_copy(rows_v.at[b], agg_sh.at[dst_v.at[chunk_of(g)]],
                             ssems[b], add=True)

        def wait_scatter(g, b):
            pltpu.make_async_copy(rows_v.at[b],
                                  agg_sh.at[dst_v.at[chunk_of(g)]],
                                  ssems[b]).wait()

        def hist(g):
            for c in range(SUBV):
                sl = pl.ds(pl.multiple_of(g * CHUNK, 8) + c * LANES, LANES)
                plsc.addupdate_scatter(deg_v, [idx_ref[sl]], ones16)

        def round_(g, bmod, drain=True, prefetch=True):
            if drain:
                wait_scatter(g - 2, (bmod + 2) % NBUF)
            if prefetch:
                gather(g + 2, (bmod + 2) % NBUF)
            wait_gather(g, bmod)
            scatter(g, bmod)
            hist(g)

        gather(0, 0)
        gather(1, 1)
        round_(0, 0, drain=False)
        round_(1, 1, drain=False)

        def outer(o, carry):
            for b in range(NBUF):
                round_(o * NBUF + 2 + b, (2 + b) % NBUF)
            return carry

        lax.fori_loop(0, (N_CHUNKS - 2 - 4) // NBUF, outer, None)
        n = N_CHUNKS
        round_(n - 4, (n - 4) % NBUF)
        round_(n - 3, (n - 3) % NBUF)
        round_(n - 2, (n - 2) % NBUF, prefetch=False)
        round_(n - 1, (n - 1) % NBUF, prefetch=False)
        wait_scatter(n - 2, (n - 2) % NBUF)
        wait_scatter(n - 1, (n - 1) % NBUF)

    @pl.when(cid == 0)
    def _():
        pipeline(x0_hbm, src_v)

    @pl.when(cid == 1)
    def _():
        pipeline(x1_hbm, dst_v)

    plsc.subcore_barrier()

    pltpu.sync_copy(deg_v, deg_out.at[cid, sid])
    cslc = pl.ds(pl.multiple_of(cid * DH, 8), DH)

    @pl.when(sid < NS - 1)
    def _():
        nslc = pl.ds(pl.multiple_of(sid * ROWS_MAIN, 8), ROWS_MAIN)
        pltpu.sync_copy(agg_sh.at[nslc], agg_out.at[nslc, cslc])

    @pl.when(sid == NS - 1)
    def _():
        nslc = pl.ds((NS - 1) * ROWS_MAIN, ROWS_LAST)
        pltpu.sync_copy(agg_sh.at[nslc], agg_out.at[nslc, cslc])


def _tc_finish_body(agg, deg, w1, b1, w2, b2, pooled_ref, logp_ref):
    ones_ns = jnp.ones((NS, 1), jnp.float32)
    ideg_col = lax.dot_general(deg[1], ones_ns, (((0,), (0,)), ((), ())),
                               preferred_element_type=jnp.float32)
    h = lax.dot_general(agg[...].astype(jnp.float32), w1[...],
                        (((1,), (1,)), ((), ())),
                        preferred_element_type=jnp.float32)
    u = jnp.maximum(h + ideg_col * b1[...], 0.0)
    s_parts = lax.dot_general(deg[0], u, (((1,), (0,)), ((), ())),
                              preferred_element_type=jnp.float32)
    s = jnp.sum(s_parts, axis=0, keepdims=True)
    pooled = lax.dot_general(s, w2[...], (((1,), (1,)), ((), ())),
                             preferred_element_type=jnp.float32)
    pooled = pooled + float(N_EDGES) * b2[...]
    pooled_ref[...] = pooled
    m = jnp.max(pooled, axis=1, keepdims=True)
    lse = m + jnp.log(jnp.sum(jnp.exp(pooled - m), axis=1, keepdims=True))
    logp_ref[...] = pooled - lse


_tc_finish = pl.pallas_call(
    _tc_finish_body,
    out_shape=(
        jax.ShapeDtypeStruct((1, D), jnp.float32),
        jax.ShapeDtypeStruct((1, D), jnp.float32),
    ),
)


@jax.jit
def kernel(x, W1, b1, W2, b2, edge_index):
    xbf = x.astype(jnp.bfloat16)
    x0 = xbf[:, :DH]
    x1 = xbf[:, DH:]
    zrows = jnp.zeros((ROWS_MAIN, DH), jnp.bfloat16)
    agg, deg_p = _sc_accumulate(x0, x1, edge_index, zrows)
    pooled, logp = _tc_finish(agg, deg_p, W1, b1.reshape(1, D),
                              W2, b2.reshape(1, D))
    return (pooled, logp)
